# Initial kernel scaffold; baseline (speedup 1.0000x reference)
import jax
import jax.numpy as jnp
from jax.experimental import pallas as pl


def kernel(node_features, edge_index, edge_type, weight, root, bias):
    raise NotImplementedError("write your pallas kernel here")


# speedup vs baseline: 5.5589x; 5.5589x over previous
import jax
import jax.numpy as jnp
from jax import lax
from jax.experimental import pallas as pl
from jax.experimental.pallas import tpu as pltpu
from jax.experimental.pallas import tpu_sc as plsc

N = 10000
E = 320000
D = 128
H = 64
NCORE = 2
NSUB = 16
EP = 327680
EROWS = EP // 128
ROWS_PER_SUB = EROWS // NSUB
K = 4
OUTER = ROWS_PER_SUB // K
AROWS = 20480
ZROWS_PER_SUB = AROWS // NSUB
KC = 8
CROWS_PER_SUB = EROWS // (NCORE * NSUB)

_SC_PARAMS = pltpu.CompilerParams(use_tc_tiling_on_sc=False)


def _sc_segment_sums(xa, srcp, dstp, typp):
    mesh = plsc.VectorSubcoreMesh(core_axis_name="c", subcore_axis_name="s")

    @pl.kernel(
        out_type=jax.ShapeDtypeStruct((NCORE, AROWS, H), jnp.float32),
        mesh=mesh,
        scratch_types=[
            pltpu.VMEM((K, 128), jnp.int32),
            pltpu.VMEM((K, 128), jnp.int32),
            pltpu.VMEM((K, 128), jnp.int32),
            pltpu.VMEM((K, 128, H), jnp.float32),
            pltpu.VMEM_SHARED((AROWS, H), jnp.float32),
        ],
        compiler_params=_SC_PARAMS,
    )
    def sc_kernel(xa_hbm, src_hbm, dst_hbm, typ_hbm, s_hbm,
                  src_i, sidx_i, typ_i, gbuf, accum):
        c = lax.axis_index("c")
        s = lax.axis_index("s")
        coff = c * N

        @pl.loop(0, 128)
        def _(r):
            @pl.loop(0, H, step=16)
            def _(k):
                gbuf[0, r, pl.ds(k, 16)] = jnp.zeros((16,), jnp.float32)

        @pl.loop(0, ZROWS_PER_SUB // 128)
        def _(z):
            pltpu.sync_copy(gbuf.at[0],
                            accum.at[pl.ds(s * ZROWS_PER_SUB + z * 128, 128)])

        plsc.subcore_barrier()

        @pl.loop(0, OUTER)
        def _(g):
            row0 = s * ROWS_PER_SUB + g * K
            pltpu.sync_copy(src_hbm.at[pl.ds(row0, K)], src_i)
            pltpu.sync_copy(dst_hbm.at[pl.ds(row0, K)], sidx_i)
            pltpu.sync_copy(typ_hbm.at[pl.ds(row0, K)], typ_i)
            for j in range(K):
                @pl.loop(0, 128, step=16)
                def _(k):
                    src_i[j, pl.ds(k, 16)] = src_i[j, pl.ds(k, 16)] + coff
                    sidx_i[j, pl.ds(k, 16)] = (
                        sidx_i[j, pl.ds(k, 16)] + typ_i[j, pl.ds(k, 16)] * N)
            for j in range(K):
                pltpu.sync_copy(xa_hbm.at[src_i.at[j]], gbuf.at[j])
                pltpu.sync_copy(gbuf.at[j], accum.at[sidx_i.at[j]], add=True)

        plsc.subcore_barrier()
        pltpu.sync_copy(
            accum.at[pl.ds(s * ZROWS_PER_SUB, ZROWS_PER_SUB)],
            s_hbm.at[c, pl.ds(s * ZROWS_PER_SUB, ZROWS_PER_SUB)])

    return sc_kernel(xa, srcp, dstp, typp)


def _sc_counts(dstp, typp):
    mesh = plsc.VectorSubcoreMesh(core_axis_name="c", subcore_axis_name="s")

    @pl.kernel(
        out_type=jax.ShapeDtypeStruct((NCORE, AROWS, 16), jnp.float32),
        mesh=mesh,
        scratch_types=[
            pltpu.VMEM((KC, 128), jnp.int32),
            pltpu.VMEM((KC, 128), jnp.int32),
            pltpu.VMEM((128, 16), jnp.float32),
            pltpu.VMEM((128, 16), jnp.float32),
            pltpu.VMEM_SHARED((AROWS, 16), jnp.float32),
        ],
        compiler_params=_SC_PARAMS,
    )
    def sc_kernel(dst_hbm, typ_hbm, c_hbm, cidx_i, typ_i, ones_b, zero_b, accum):
        c = lax.axis_index("c")
        s = lax.axis_index("s")

        @pl.loop(0, 128)
        def _(r):
            ones_b[r, pl.ds(0, 16)] = jnp.ones((16,), jnp.float32)
            zero_b[r, pl.ds(0, 16)] = jnp.zeros((16,), jnp.float32)

        @pl.loop(0, ZROWS_PER_SUB // 128)
        def _(z):
            pltpu.sync_copy(zero_b,
                            accum.at[pl.ds(s * ZROWS_PER_SUB + z * 128, 128)])

        plsc.subcore_barrier()

        @pl.loop(0, CROWS_PER_SUB // KC)
        def _(g):
            row0 = (c * NSUB + s) * CROWS_PER_SUB + g * KC
            pltpu.sync_copy(dst_hbm.at[pl.ds(row0, KC)], cidx_i)
            pltpu.sync_copy(typ_hbm.at[pl.ds(row0, KC)], typ_i)
            for j in range(KC):
                @pl.loop(0, 128, step=16)
                def _(k):
                    cidx_i[j, pl.ds(k, 16)] = (
                        cidx_i[j, pl.ds(k, 16)] + typ_i[j, pl.ds(k, 16)] * N)
            for j in range(KC):
                pltpu.sync_copy(ones_b, accum.at[cidx_i.at[j]], add=True)

        plsc.subcore_barrier()
        pltpu.sync_copy(
            accum.at[pl.ds(s * ZROWS_PER_SUB, ZROWS_PER_SUB)],
            c_hbm.at[c, pl.ds(s * ZROWS_PER_SUB, ZROWS_PER_SUB)])

    return sc_kernel(dstp, typp)


def _tc_combine(x, S, C, root, weight, bias2d):
    B = 1000
    grid = (N // B,)
    NB = N // B

    def tc_body(x_ref, s00, s10, s01, s11, c00, c10, c01, c11,
                root_ref, w_ref, b_ref, o_ref):
        cnt0 = c00[0, :, 0:1] + c10[0, :, 0:1]
        cnt1 = c01[0, :, 0:1] + c11[0, :, 0:1]
        inv0 = 1.0 / jnp.clip(cnt0, 1.0, None)
        inv1 = 1.0 / jnp.clip(cnt1, 1.0, None)
        m0 = jnp.concatenate([s00[0], s10[0]], axis=1) * inv0
        m1 = jnp.concatenate([s01[0], s11[0]], axis=1) * inv1
        out = jnp.dot(x_ref[...], root_ref[...],
                      preferred_element_type=jnp.float32)
        out = out + jnp.dot(m0, w_ref[0], preferred_element_type=jnp.float32)
        out = out + jnp.dot(m1, w_ref[1], preferred_element_type=jnp.float32)
        o_ref[...] = out + b_ref[...]

    sblk = (1, B, H)
    cblk = (1, B, 16)
    return pl.pallas_call(
        tc_body,
        grid=grid,
        in_specs=[
            pl.BlockSpec((B, D), lambda i: (i, 0)),
            pl.BlockSpec(sblk, lambda i: (0, i, 0)),
            pl.BlockSpec(sblk, lambda i: (1, i, 0)),
            pl.BlockSpec(sblk, lambda i: (0, i + NB, 0)),
            pl.BlockSpec(sblk, lambda i: (1, i + NB, 0)),
            pl.BlockSpec(cblk, lambda i: (0, i, 0)),
            pl.BlockSpec(cblk, lambda i: (1, i, 0)),
            pl.BlockSpec(cblk, lambda i: (0, i + NB, 0)),
            pl.BlockSpec(cblk, lambda i: (1, i + NB, 0)),
            pl.BlockSpec((D, D), lambda i: (0, 0)),
            pl.BlockSpec((2, D, D), lambda i: (0, 0, 0)),
            pl.BlockSpec((1, D), lambda i: (0, 0)),
        ],
        out_specs=pl.BlockSpec((B, D), lambda i: (i, 0)),
        out_shape=jax.ShapeDtypeStruct((N, D), jnp.float32),
    )(x, S, S, S, S, C, C, C, C, root, weight, bias2d)


def kernel(node_features, edge_index, edge_type, weight, root, bias):
    x = node_features
    src = edge_index[0]
    dst = edge_index[1]

    xa = jnp.concatenate([x[:, :H], x[:, H:]], axis=0)

    pad = EP - E
    srcp = jnp.concatenate([src, jnp.zeros((pad,), jnp.int32)]).reshape(EROWS, 128)
    dstp = jnp.concatenate([dst, jnp.full((pad,), N, jnp.int32)]).reshape(EROWS, 128)
    typp = jnp.concatenate([edge_type, jnp.full((pad,), 1, jnp.int32)]).reshape(EROWS, 128)

    S = _sc_segment_sums(xa, srcp, dstp, typp)
    C = _sc_counts(dstp, typp)
    return _tc_combine(x, S, C, root, weight, bias.reshape(1, D))


# speedup vs baseline: 6.9095x; 1.2430x over previous
import jax
import jax.numpy as jnp
from jax import lax
from jax.experimental import pallas as pl
from jax.experimental.pallas import tpu as pltpu
from jax.experimental.pallas import tpu_sc as plsc

N = 10000
E = 320000
D = 128
H = 64
NCORE = 2
NSUB = 16
EP = 327680
EROWS = EP // 128
ROWS_PER_SUB = EROWS // NSUB
K = 2
OUTER = ROWS_PER_SUB // K
AROWS = 20480
ZROWS_PER_SUB = AROWS // NSUB
KC = 8
CROWS_PER_SUB = EROWS // (NCORE * NSUB)

_SC_PARAMS = pltpu.CompilerParams(use_tc_tiling_on_sc=False)


def _sc_segment_sums(xa, srcp, dstp, typp):
    mesh = plsc.VectorSubcoreMesh(core_axis_name="c", subcore_axis_name="s")

    @pl.kernel(
        out_type=jax.ShapeDtypeStruct((NCORE, AROWS, H), jnp.float32),
        mesh=mesh,
        scratch_types=[
            pltpu.VMEM((2, K, 128), jnp.int32),
            pltpu.VMEM((2, K, 128), jnp.int32),
            pltpu.VMEM((2, K, 128), jnp.int32),
            pltpu.VMEM((2, K, 128, H), jnp.float32),
            pltpu.VMEM_SHARED((AROWS, H), jnp.float32),
            pltpu.SemaphoreType.DMA((2, K)),
        ],
        compiler_params=_SC_PARAMS,
    )
    def sc_kernel(xa_hbm, src_hbm, dst_hbm, typ_hbm, s_hbm,
                  src_i, sidx_i, typ_i, gbuf, accum, gsem):
        c = lax.axis_index("c")
        s = lax.axis_index("s")
        coff = c * N

        @pl.loop(0, 128)
        def _(r):
            @pl.loop(0, H, step=16)
            def _(k):
                gbuf[0, 0, r, pl.ds(k, 16)] = jnp.zeros((16,), jnp.float32)

        @pl.loop(0, ZROWS_PER_SUB // 128)
        def _(z):
            pltpu.sync_copy(gbuf.at[0, 0],
                            accum.at[pl.ds(s * ZROWS_PER_SUB + z * 128, 128)])

        plsc.subcore_barrier()

        def fire(p, g):
            row0 = s * ROWS_PER_SUB + g * K
            pltpu.sync_copy(src_hbm.at[pl.ds(row0, K)], src_i.at[p])
            pltpu.sync_copy(dst_hbm.at[pl.ds(row0, K)], sidx_i.at[p])
            pltpu.sync_copy(typ_hbm.at[pl.ds(row0, K)], typ_i.at[p])
            for j in range(K):
                @pl.loop(0, 128, step=16)
                def _(k):
                    src_i[p, j, pl.ds(k, 16)] = src_i[p, j, pl.ds(k, 16)] + coff
                    sidx_i[p, j, pl.ds(k, 16)] = (
                        sidx_i[p, j, pl.ds(k, 16)]
                        + typ_i[p, j, pl.ds(k, 16)] * N)
            for j in range(K):
                pltpu.async_copy(xa_hbm.at[src_i.at[p, j]], gbuf.at[p, j],
                                 gsem.at[p, j])

        def drain(p):
            for j in range(K):
                pltpu.make_async_copy(xa_hbm.at[src_i.at[p, j]],
                                      gbuf.at[p, j], gsem.at[p, j]).wait()
                pltpu.sync_copy(gbuf.at[p, j], accum.at[sidx_i.at[p, j]],
                                add=True)

        fire(0, 0)

        @pl.loop(0, OUTER // 2)
        def _(g2):
            fire(1, 2 * g2 + 1)
            drain(0)

            @pl.when(g2 < OUTER // 2 - 1)
            def _():
                fire(0, 2 * g2 + 2)

            drain(1)

        plsc.subcore_barrier()
        pltpu.sync_copy(
            accum.at[pl.ds(s * ZROWS_PER_SUB, ZROWS_PER_SUB)],
            s_hbm.at[c, pl.ds(s * ZROWS_PER_SUB, ZROWS_PER_SUB)])

    return sc_kernel(xa, srcp, dstp, typp)


def _sc_counts(dstp, typp):
    mesh = plsc.VectorSubcoreMesh(core_axis_name="c", subcore_axis_name="s")

    @pl.kernel(
        out_type=jax.ShapeDtypeStruct((NCORE, AROWS, 16), jnp.float32),
        mesh=mesh,
        scratch_types=[
            pltpu.VMEM((KC, 128), jnp.int32),
            pltpu.VMEM((KC, 128), jnp.int32),
            pltpu.VMEM((128, 16), jnp.float32),
            pltpu.VMEM((128, 16), jnp.float32),
            pltpu.VMEM_SHARED((AROWS, 16), jnp.float32),
        ],
        compiler_params=_SC_PARAMS,
    )
    def sc_kernel(dst_hbm, typ_hbm, c_hbm, cidx_i, typ_i, ones_b, zero_b, accum):
        c = lax.axis_index("c")
        s = lax.axis_index("s")

        @pl.loop(0, 128)
        def _(r):
            ones_b[r, pl.ds(0, 16)] = jnp.ones((16,), jnp.float32)
            zero_b[r, pl.ds(0, 16)] = jnp.zeros((16,), jnp.float32)

        @pl.loop(0, ZROWS_PER_SUB // 128)
        def _(z):
            pltpu.sync_copy(zero_b,
                            accum.at[pl.ds(s * ZROWS_PER_SUB + z * 128, 128)])

        plsc.subcore_barrier()

        @pl.loop(0, CROWS_PER_SUB // KC)
        def _(g):
            row0 = (c * NSUB + s) * CROWS_PER_SUB + g * KC
            pltpu.sync_copy(dst_hbm.at[pl.ds(row0, KC)], cidx_i)
            pltpu.sync_copy(typ_hbm.at[pl.ds(row0, KC)], typ_i)
            for j in range(KC):
                @pl.loop(0, 128, step=16)
                def _(k):
                    cidx_i[j, pl.ds(k, 16)] = (
                        cidx_i[j, pl.ds(k, 16)] + typ_i[j, pl.ds(k, 16)] * N)
            for j in range(KC):
                pltpu.sync_copy(ones_b, accum.at[cidx_i.at[j]], add=True)

        plsc.subcore_barrier()
        pltpu.sync_copy(
            accum.at[pl.ds(s * ZROWS_PER_SUB, ZROWS_PER_SUB)],
            c_hbm.at[c, pl.ds(s * ZROWS_PER_SUB, ZROWS_PER_SUB)])

    return sc_kernel(dstp, typp)


def _tc_combine(x, S, C, root, weight, bias2d):
    B = 1000
    grid = (N // B,)
    NB = N // B

    def tc_body(x_ref, s00, s10, s01, s11, c00, c10, c01, c11,
                root_ref, w_ref, b_ref, o_ref):
        cnt0 = c00[0, :, 0:1] + c10[0, :, 0:1]
        cnt1 = c01[0, :, 0:1] + c11[0, :, 0:1]
        inv0 = 1.0 / jnp.clip(cnt0, 1.0, None)
        inv1 = 1.0 / jnp.clip(cnt1, 1.0, None)
        m0 = jnp.concatenate([s00[0], s10[0]], axis=1) * inv0
        m1 = jnp.concatenate([s01[0], s11[0]], axis=1) * inv1
        out = jnp.dot(x_ref[...], root_ref[...],
                      preferred_element_type=jnp.float32)
        out = out + jnp.dot(m0, w_ref[0], preferred_element_type=jnp.float32)
        out = out + jnp.dot(m1, w_ref[1], preferred_element_type=jnp.float32)
        o_ref[...] = out + b_ref[...]

    sblk = (1, B, H)
    cblk = (1, B, 16)
    return pl.pallas_call(
        tc_body,
        grid=grid,
        in_specs=[
            pl.BlockSpec((B, D), lambda i: (i, 0)),
            pl.BlockSpec(sblk, lambda i: (0, i, 0)),
            pl.BlockSpec(sblk, lambda i: (1, i, 0)),
            pl.BlockSpec(sblk, lambda i: (0, i + NB, 0)),
            pl.BlockSpec(sblk, lambda i: (1, i + NB, 0)),
            pl.BlockSpec(cblk, lambda i: (0, i, 0)),
            pl.BlockSpec(cblk, lambda i: (1, i, 0)),
            pl.BlockSpec(cblk, lambda i: (0, i + NB, 0)),
            pl.BlockSpec(cblk, lambda i: (1, i + NB, 0)),
            pl.BlockSpec((D, D), lambda i: (0, 0)),
            pl.BlockSpec((2, D, D), lambda i: (0, 0, 0)),
            pl.BlockSpec((1, D), lambda i: (0, 0)),
        ],
        out_specs=pl.BlockSpec((B, D), lambda i: (i, 0)),
        out_shape=jax.ShapeDtypeStruct((N, D), jnp.float32),
    )(x, S, S, S, S, C, C, C, C, root, weight, bias2d)


def kernel(node_features, edge_index, edge_type, weight, root, bias):
    x = node_features
    src = edge_index[0]
    dst = edge_index[1]

    xa = jnp.concatenate([x[:, :H], x[:, H:]], axis=0)

    pad = EP - E
    srcp = jnp.concatenate([src, jnp.zeros((pad,), jnp.int32)]).reshape(EROWS, 128)
    dstp = jnp.concatenate([dst, jnp.full((pad,), N, jnp.int32)]).reshape(EROWS, 128)
    typp = jnp.concatenate([edge_type, jnp.full((pad,), 1, jnp.int32)]).reshape(EROWS, 128)

    S = _sc_segment_sums(xa, srcp, dstp, typp)
    C = _sc_counts(dstp, typp)
    return _tc_combine(x, S, C, root, weight, bias.reshape(1, D))


# speedup vs baseline: 7.7572x; 1.1227x over previous
import jax
import jax.numpy as jnp
from jax import lax
from jax.experimental import pallas as pl
from jax.experimental.pallas import tpu as pltpu
from jax.experimental.pallas import tpu_sc as plsc

N = 10000
E = 320000
D = 128
H = 64
NCORE = 2
NSUB = 16
EP = 327680
EROWS = EP // 128
ROWS_PER_SUB = EROWS // NSUB
CH = 16
NCHUNK = ROWS_PER_SUB // CH
RING = 4
AROWS = 20480
ZROWS_PER_SUB = AROWS // NSUB
KC = 8
CROWS_PER_SUB = EROWS // (NCORE * NSUB)

_SC_PARAMS = pltpu.CompilerParams(use_tc_tiling_on_sc=False)


def _sc_segment_sums(xa, srcp, dstp, typp):
    mesh = plsc.VectorSubcoreMesh(core_axis_name="c", subcore_axis_name="s")

    @pl.kernel(
        out_type=jax.ShapeDtypeStruct((NCORE, AROWS, H), jnp.float32),
        mesh=mesh,
        scratch_types=[
            pltpu.VMEM((2, CH, 128), jnp.int32),
            pltpu.VMEM((2, CH, 128), jnp.int32),
            pltpu.VMEM((2, CH, 128), jnp.int32),
            pltpu.VMEM((RING, 128, H), jnp.float32),
            pltpu.VMEM_SHARED((AROWS, H), jnp.float32),
            pltpu.SemaphoreType.DMA((2,)),
            pltpu.SemaphoreType.DMA((RING,)),
        ],
        compiler_params=_SC_PARAMS,
    )
    def sc_kernel(xa_hbm, src_hbm, dst_hbm, typ_hbm, s_hbm,
                  srcb, sidxb, typb, gbuf, accum, isem, gsem):
        c = lax.axis_index("c")
        s = lax.axis_index("s")
        coff = c * N

        @pl.loop(0, 128)
        def _(r):
            @pl.loop(0, H, step=16)
            def _(k):
                gbuf[0, r, pl.ds(k, 16)] = jnp.zeros((16,), jnp.float32)

        @pl.loop(0, ZROWS_PER_SUB // 128)
        def _(z):
            pltpu.sync_copy(gbuf.at[0],
                            accum.at[pl.ds(s * ZROWS_PER_SUB + z * 128, 128)])

        plsc.subcore_barrier()

        def prefetch(p, ch):
            row0 = s * ROWS_PER_SUB + ch * CH
            pltpu.async_copy(src_hbm.at[pl.ds(row0, CH)], srcb.at[p], isem.at[p])
            pltpu.async_copy(dst_hbm.at[pl.ds(row0, CH)], sidxb.at[p], isem.at[p])
            pltpu.async_copy(typ_hbm.at[pl.ds(row0, CH)], typb.at[p], isem.at[p])

        def wait_idx(p):
            for _ in range(3):
                pltpu.make_async_copy(src_hbm.at[pl.ds(0, CH)], srcb.at[p],
                                      isem.at[p]).wait()

        def drain(slot, p, row):
            pltpu.make_async_copy(xa_hbm.at[srcb.at[p, row]], gbuf.at[slot],
                                  gsem.at[slot]).wait()
            pltpu.sync_copy(gbuf.at[slot], accum.at[sidxb.at[p, row]], add=True)

        def chunk_body(p):
            wait_idx(p)
            for r in range(CH):
                @pl.loop(0, 128, step=16)
                def _(k):
                    srcb[p, r, pl.ds(k, 16)] = srcb[p, r, pl.ds(k, 16)] + coff
                    sidxb[p, r, pl.ds(k, 16)] = (
                        sidxb[p, r, pl.ds(k, 16)]
                        + typb[p, r, pl.ds(k, 16)] * N)
                if r >= RING:
                    drain(r % RING, p, r - RING)
                pltpu.async_copy(xa_hbm.at[srcb.at[p, r]], gbuf.at[r % RING],
                                 gsem.at[r % RING])
            for r in range(CH - RING, CH):
                drain(r % RING, p, r)

        prefetch(0, 0)

        @pl.loop(0, NCHUNK // 2)
        def _(ch2):
            prefetch(1, 2 * ch2 + 1)
            chunk_body(0)

            @pl.when(ch2 < NCHUNK // 2 - 1)
            def _():
                prefetch(0, 2 * ch2 + 2)

            chunk_body(1)

        plsc.subcore_barrier()
        pltpu.sync_copy(
            accum.at[pl.ds(s * ZROWS_PER_SUB, ZROWS_PER_SUB)],
            s_hbm.at[c, pl.ds(s * ZROWS_PER_SUB, ZROWS_PER_SUB)])

    return sc_kernel(xa, srcp, dstp, typp)


def _sc_counts(dstp, typp):
    mesh = plsc.VectorSubcoreMesh(core_axis_name="c", subcore_axis_name="s")

    @pl.kernel(
        out_type=jax.ShapeDtypeStruct((NCORE, AROWS, 16), jnp.float32),
        mesh=mesh,
        scratch_types=[
            pltpu.VMEM((KC, 128), jnp.int32),
            pltpu.VMEM((KC, 128), jnp.int32),
            pltpu.VMEM((128, 16), jnp.float32),
            pltpu.VMEM((128, 16), jnp.float32),
            pltpu.VMEM_SHARED((AROWS, 16), jnp.float32),
        ],
        compiler_params=_SC_PARAMS,
    )
    def sc_kernel(dst_hbm, typ_hbm, c_hbm, cidx_i, typ_i, ones_b, zero_b, accum):
        c = lax.axis_index("c")
        s = lax.axis_index("s")

        @pl.loop(0, 128)
        def _(r):
            ones_b[r, pl.ds(0, 16)] = jnp.ones((16,), jnp.float32)
            zero_b[r, pl.ds(0, 16)] = jnp.zeros((16,), jnp.float32)

        @pl.loop(0, ZROWS_PER_SUB // 128)
        def _(z):
            pltpu.sync_copy(zero_b,
                            accum.at[pl.ds(s * ZROWS_PER_SUB + z * 128, 128)])

        plsc.subcore_barrier()

        @pl.loop(0, CROWS_PER_SUB // KC)
        def _(g):
            row0 = (c * NSUB + s) * CROWS_PER_SUB + g * KC
            pltpu.sync_copy(dst_hbm.at[pl.ds(row0, KC)], cidx_i)
            pltpu.sync_copy(typ_hbm.at[pl.ds(row0, KC)], typ_i)
            for j in range(KC):
                @pl.loop(0, 128, step=16)
                def _(k):
                    cidx_i[j, pl.ds(k, 16)] = (
                        cidx_i[j, pl.ds(k, 16)] + typ_i[j, pl.ds(k, 16)] * N)
            for j in range(KC):
                pltpu.sync_copy(ones_b, accum.at[cidx_i.at[j]], add=True)

        plsc.subcore_barrier()
        pltpu.sync_copy(
            accum.at[pl.ds(s * ZROWS_PER_SUB, ZROWS_PER_SUB)],
            c_hbm.at[c, pl.ds(s * ZROWS_PER_SUB, ZROWS_PER_SUB)])

    return sc_kernel(dstp, typp)


def _tc_combine(x, S, C, root, weight, bias2d):
    B = 1000
    grid = (N // B,)
    NB = N // B

    def tc_body(x_ref, s00, s10, s01, s11, c00, c10, c01, c11,
                root_ref, w_ref, b_ref, o_ref):
        cnt0 = c00[0, :, 0:1] + c10[0, :, 0:1]
        cnt1 = c01[0, :, 0:1] + c11[0, :, 0:1]
        inv0 = 1.0 / jnp.clip(cnt0, 1.0, None)
        inv1 = 1.0 / jnp.clip(cnt1, 1.0, None)
        m0 = jnp.concatenate([s00[0], s10[0]], axis=1) * inv0
        m1 = jnp.concatenate([s01[0], s11[0]], axis=1) * inv1
        out = jnp.dot(x_ref[...], root_ref[...],
                      preferred_element_type=jnp.float32)
        out = out + jnp.dot(m0, w_ref[0], preferred_element_type=jnp.float32)
        out = out + jnp.dot(m1, w_ref[1], preferred_element_type=jnp.float32)
        o_ref[...] = out + b_ref[...]

    sblk = (1, B, H)
    cblk = (1, B, 16)
    return pl.pallas_call(
        tc_body,
        grid=grid,
        in_specs=[
            pl.BlockSpec((B, D), lambda i: (i, 0)),
            pl.BlockSpec(sblk, lambda i: (0, i, 0)),
            pl.BlockSpec(sblk, lambda i: (1, i, 0)),
            pl.BlockSpec(sblk, lambda i: (0, i + NB, 0)),
            pl.BlockSpec(sblk, lambda i: (1, i + NB, 0)),
            pl.BlockSpec(cblk, lambda i: (0, i, 0)),
            pl.BlockSpec(cblk, lambda i: (1, i, 0)),
            pl.BlockSpec(cblk, lambda i: (0, i + NB, 0)),
            pl.BlockSpec(cblk, lambda i: (1, i + NB, 0)),
            pl.BlockSpec((D, D), lambda i: (0, 0)),
            pl.BlockSpec((2, D, D), lambda i: (0, 0, 0)),
            pl.BlockSpec((1, D), lambda i: (0, 0)),
        ],
        out_specs=pl.BlockSpec((B, D), lambda i: (i, 0)),
        out_shape=jax.ShapeDtypeStruct((N, D), jnp.float32),
    )(x, S, S, S, S, C, C, C, C, root, weight, bias2d)


def kernel(node_features, edge_index, edge_type, weight, root, bias):
    x = node_features
    src = edge_index[0]
    dst = edge_index[1]

    xa = jnp.concatenate([x[:, :H], x[:, H:]], axis=0)

    pad = EP - E
    srcp = jnp.concatenate([src, jnp.zeros((pad,), jnp.int32)]).reshape(EROWS, 128)
    dstp = jnp.concatenate([dst, jnp.full((pad,), N, jnp.int32)]).reshape(EROWS, 128)
    typp = jnp.concatenate([edge_type, jnp.full((pad,), 1, jnp.int32)]).reshape(EROWS, 128)

    S = _sc_segment_sums(xa, srcp, dstp, typp)
    C = _sc_counts(dstp, typp)
    return _tc_combine(x, S, C, root, weight, bias.reshape(1, D))


# speedup vs baseline: 7.9719x; 1.0277x over previous
import jax
import jax.numpy as jnp
from jax import lax
from jax.experimental import pallas as pl
from jax.experimental.pallas import tpu as pltpu
from jax.experimental.pallas import tpu_sc as plsc

N = 10000
E = 320000
D = 128
H = 64
NCORE = 2
NSUB = 16
EP = 327680
EROWS = EP // 128
ROWS_PER_SUB = EROWS // NSUB
CH = 16
NCHUNK = ROWS_PER_SUB // CH
RING = 4
AROWS = 20480
ZROWS_PER_SUB = AROWS // NSUB
KC = 8
CROWS_PER_SUB = EROWS // (NCORE * NSUB)

_SC_PARAMS = pltpu.CompilerParams(use_tc_tiling_on_sc=False)


def _sc_segment_sums(xa, srcp, dstp, typp):
    mesh = plsc.VectorSubcoreMesh(core_axis_name="c", subcore_axis_name="s")

    @pl.kernel(
        out_type=jax.ShapeDtypeStruct((NCORE, AROWS, H), jnp.float32),
        mesh=mesh,
        scratch_types=[
            pltpu.VMEM((2, CH, 128), jnp.int32),
            pltpu.VMEM((2, CH, 128), jnp.int32),
            pltpu.VMEM((2, CH, 128), jnp.int32),
            pltpu.VMEM((RING, 128, H), jnp.float32),
            pltpu.VMEM_SHARED((AROWS, H), jnp.float32),
            pltpu.SemaphoreType.DMA((2,)),
            pltpu.SemaphoreType.DMA((RING,)),
        ],
        compiler_params=_SC_PARAMS,
    )
    def sc_kernel(xa_hbm, src_hbm, dst_hbm, typ_hbm, s_hbm,
                  srcb, sidxb, typb, gbuf, accum, isem, gsem):
        c = lax.axis_index("c")
        s = lax.axis_index("s")
        coff = c * N

        @pl.loop(0, 128)
        def _(r):
            @pl.loop(0, H, step=16)
            def _(k):
                gbuf[0, r, pl.ds(k, 16)] = jnp.zeros((16,), jnp.float32)

        for z in range(ZROWS_PER_SUB // 128):
            pltpu.async_copy(gbuf.at[0],
                             accum.at[pl.ds(s * ZROWS_PER_SUB + z * 128, 128)],
                             gsem.at[0])
        for z in range(ZROWS_PER_SUB // 128):
            pltpu.make_async_copy(
                gbuf.at[0], accum.at[pl.ds(s * ZROWS_PER_SUB + z * 128, 128)],
                gsem.at[0]).wait()

        plsc.subcore_barrier()

        def prefetch(p, ch):
            row0 = s * ROWS_PER_SUB + ch * CH
            pltpu.async_copy(src_hbm.at[pl.ds(row0, CH)], srcb.at[p], isem.at[p])
            pltpu.async_copy(dst_hbm.at[pl.ds(row0, CH)], sidxb.at[p], isem.at[p])
            pltpu.async_copy(typ_hbm.at[pl.ds(row0, CH)], typb.at[p], isem.at[p])

        def wait_idx(p):
            for _ in range(3):
                pltpu.make_async_copy(src_hbm.at[pl.ds(0, CH)], srcb.at[p],
                                      isem.at[p]).wait()

        def drain(slot, p, row):
            pltpu.make_async_copy(xa_hbm.at[srcb.at[p, row]], gbuf.at[slot],
                                  gsem.at[slot]).wait()
            pltpu.sync_copy(gbuf.at[slot], accum.at[sidxb.at[p, row]], add=True)

        def chunk_body(p):
            wait_idx(p)
            for r in range(CH):
                @pl.loop(0, 128, step=16)
                def _(k):
                    srcb[p, r, pl.ds(k, 16)] = srcb[p, r, pl.ds(k, 16)] + coff
                    sidxb[p, r, pl.ds(k, 16)] = (
                        sidxb[p, r, pl.ds(k, 16)]
                        + typb[p, r, pl.ds(k, 16)] * N)
                if r >= RING:
                    drain(r % RING, p, r - RING)
                pltpu.async_copy(xa_hbm.at[srcb.at[p, r]],
                                 gbuf.at[r % RING], gsem.at[r % RING])
            for r in range(CH - RING, CH):
                drain(r % RING, p, r)

        prefetch(0, 0)

        @pl.loop(0, NCHUNK // 2)
        def _(ch2):
            prefetch(1, 2 * ch2 + 1)
            chunk_body(0)

            @pl.when(ch2 < NCHUNK // 2 - 1)
            def _():
                prefetch(0, 2 * ch2 + 2)

            chunk_body(1)

        plsc.subcore_barrier()
        pltpu.sync_copy(
            accum.at[pl.ds(s * ZROWS_PER_SUB, ZROWS_PER_SUB)],
            s_hbm.at[c, pl.ds(s * ZROWS_PER_SUB, ZROWS_PER_SUB)])

    return sc_kernel(xa, srcp, dstp, typp)


def _sc_counts(dstp, typp):
    mesh = plsc.VectorSubcoreMesh(core_axis_name="c", subcore_axis_name="s")

    @pl.kernel(
        out_type=jax.ShapeDtypeStruct((NCORE, AROWS, 16), jnp.float32),
        mesh=mesh,
        scratch_types=[
            pltpu.VMEM((CROWS_PER_SUB, 128), jnp.int32),
            pltpu.VMEM((CROWS_PER_SUB, 128), jnp.int32),
            pltpu.VMEM((128, 16), jnp.float32),
            pltpu.VMEM((128, 16), jnp.float32),
            pltpu.VMEM_SHARED((AROWS, 16), jnp.float32),
            pltpu.SemaphoreType.DMA,
            pltpu.SemaphoreType.DMA,
        ],
        compiler_params=_SC_PARAMS,
    )
    def sc_kernel(dst_hbm, typ_hbm, c_hbm, cidx_i, typ_i, ones_b, zero_b, accum,
                  isem, ssem):
        c = lax.axis_index("c")
        s = lax.axis_index("s")
        row0 = (c * NSUB + s) * CROWS_PER_SUB

        pltpu.async_copy(dst_hbm.at[pl.ds(row0, CROWS_PER_SUB)], cidx_i, isem)
        pltpu.async_copy(typ_hbm.at[pl.ds(row0, CROWS_PER_SUB)], typ_i, isem)

        @pl.loop(0, 128)
        def _(r):
            ones_b[r, pl.ds(0, 16)] = jnp.ones((16,), jnp.float32)
            zero_b[r, pl.ds(0, 16)] = jnp.zeros((16,), jnp.float32)

        for z in range(ZROWS_PER_SUB // 128):
            pltpu.async_copy(
                zero_b, accum.at[pl.ds(s * ZROWS_PER_SUB + z * 128, 128)], ssem)
        for z in range(ZROWS_PER_SUB // 128):
            pltpu.make_async_copy(
                zero_b, accum.at[pl.ds(s * ZROWS_PER_SUB + z * 128, 128)],
                ssem).wait()

        for _ in range(2):
            pltpu.make_async_copy(dst_hbm.at[pl.ds(0, CROWS_PER_SUB)], cidx_i,
                                  isem).wait()

        plsc.subcore_barrier()

        @pl.loop(0, CROWS_PER_SUB)
        def _(g):
            @pl.loop(0, 128, step=16)
            def _(k):
                cidx_i[g, pl.ds(k, 16)] = (
                    cidx_i[g, pl.ds(k, 16)] + typ_i[g, pl.ds(k, 16)] * N)
            pltpu.async_copy(ones_b, accum.at[cidx_i.at[g]], ssem, add=True)

        @pl.loop(0, CROWS_PER_SUB)
        def _(g):
            pltpu.make_async_copy(ones_b, accum.at[cidx_i.at[g]], ssem).wait()

        plsc.subcore_barrier()
        pltpu.sync_copy(
            accum.at[pl.ds(s * ZROWS_PER_SUB, ZROWS_PER_SUB)],
            c_hbm.at[c, pl.ds(s * ZROWS_PER_SUB, ZROWS_PER_SUB)])

    return sc_kernel(dstp, typp)


def _tc_combine(x, S, C, root, weight, bias2d):
    B = 1000
    grid = (N // B,)
    NB = N // B

    def tc_body(x_ref, s00, s10, s01, s11, c00, c10, c01, c11,
                root_ref, w_ref, b_ref, o_ref):
        cnt0 = c00[0, :, 0:1] + c10[0, :, 0:1]
        cnt1 = c01[0, :, 0:1] + c11[0, :, 0:1]
        inv0 = 1.0 / jnp.clip(cnt0, 1.0, None)
        inv1 = 1.0 / jnp.clip(cnt1, 1.0, None)
        m0 = jnp.concatenate([s00[0], s10[0]], axis=1) * inv0
        m1 = jnp.concatenate([s01[0], s11[0]], axis=1) * inv1
        out = jnp.dot(x_ref[...], root_ref[...],
                      preferred_element_type=jnp.float32)
        out = out + jnp.dot(m0, w_ref[0], preferred_element_type=jnp.float32)
        out = out + jnp.dot(m1, w_ref[1], preferred_element_type=jnp.float32)
        o_ref[...] = out + b_ref[...]

    sblk = (1, B, H)
    cblk = (1, B, 16)
    return pl.pallas_call(
        tc_body,
        grid=grid,
        in_specs=[
            pl.BlockSpec((B, D), lambda i: (i, 0)),
            pl.BlockSpec(sblk, lambda i: (0, i, 0)),
            pl.BlockSpec(sblk, lambda i: (1, i, 0)),
            pl.BlockSpec(sblk, lambda i: (0, i + NB, 0)),
            pl.BlockSpec(sblk, lambda i: (1, i + NB, 0)),
            pl.BlockSpec(cblk, lambda i: (0, i, 0)),
            pl.BlockSpec(cblk, lambda i: (1, i, 0)),
            pl.BlockSpec(cblk, lambda i: (0, i + NB, 0)),
            pl.BlockSpec(cblk, lambda i: (1, i + NB, 0)),
            pl.BlockSpec((D, D), lambda i: (0, 0)),
            pl.BlockSpec((2, D, D), lambda i: (0, 0, 0)),
            pl.BlockSpec((1, D), lambda i: (0, 0)),
        ],
        out_specs=pl.BlockSpec((B, D), lambda i: (i, 0)),
        out_shape=jax.ShapeDtypeStruct((N, D), jnp.float32),
    )(x, S, S, S, S, C, C, C, C, root, weight, bias2d)


def kernel(node_features, edge_index, edge_type, weight, root, bias):
    x = node_features
    src = edge_index[0]
    dst = edge_index[1]

    xa = jnp.concatenate([x[:, :H], x[:, H:]], axis=0)

    pad = EP - E
    srcp = jnp.concatenate([src, jnp.zeros((pad,), jnp.int32)]).reshape(EROWS, 128)
    dstp = jnp.concatenate([dst, jnp.full((pad,), N, jnp.int32)]).reshape(EROWS, 128)
    typp = jnp.concatenate([edge_type, jnp.full((pad,), 1, jnp.int32)]).reshape(EROWS, 128)

    S = _sc_segment_sums(xa, srcp, dstp, typp)
    C = _sc_counts(dstp, typp)
    return _tc_combine(x, S, C, root, weight, bias.reshape(1, D))


# speedup vs baseline: 8.0868x; 1.0144x over previous
import jax
import jax.numpy as jnp
from jax import lax
from jax.experimental import pallas as pl
from jax.experimental.pallas import tpu as pltpu
from jax.experimental.pallas import tpu_sc as plsc

N = 10000
E = 320000
D = 128
H = 64
NCORE = 2
NSUB = 16
EP = 327680
EROWS = EP // 128
ROWS_PER_SUB = EROWS // NSUB
CH = 16
NCHUNK = ROWS_PER_SUB // CH
RING = 4
AROWS = 20480
ZROWS_PER_SUB = AROWS // NSUB
KC = 8
CROWS_PER_SUB = EROWS // (NCORE * NSUB)

_SC_PARAMS = pltpu.CompilerParams(use_tc_tiling_on_sc=False,
                                  needs_layout_passes=False)


def _sc_segment_sums(xa, srcp, dstp, typp):
    mesh = plsc.VectorSubcoreMesh(core_axis_name="c", subcore_axis_name="s")

    @pl.kernel(
        out_type=jax.ShapeDtypeStruct((NCORE, AROWS, H), jnp.float32),
        mesh=mesh,
        scratch_types=[
            pltpu.VMEM((2, CH, 128), jnp.int32),
            pltpu.VMEM((2, CH, 128), jnp.int32),
            pltpu.VMEM((2, CH, 128), jnp.int32),
            pltpu.VMEM((RING, 128, H), jnp.bfloat16),
            pltpu.VMEM((128, H), jnp.float32),
            pltpu.VMEM_SHARED((AROWS, H), jnp.float32),
            pltpu.SemaphoreType.DMA((2,)),
            pltpu.SemaphoreType.DMA((RING,)),
        ],
        compiler_params=_SC_PARAMS,
    )
    def sc_kernel(xa_hbm, src_hbm, dst_hbm, typ_hbm, s_hbm,
                  srcb, sidxb, typb, gbuf, ubuf, accum, isem, gsem):
        c = lax.axis_index("c")
        s = lax.axis_index("s")
        coff = c * N

        @pl.loop(0, 128)
        def _(r):
            @pl.loop(0, H, step=16)
            def _(k):
                ubuf[r, pl.ds(k, 16)] = jnp.zeros((16,), jnp.float32)

        for z in range(ZROWS_PER_SUB // 128):
            pltpu.async_copy(ubuf,
                             accum.at[pl.ds(s * ZROWS_PER_SUB + z * 128, 128)],
                             gsem.at[0])
        for z in range(ZROWS_PER_SUB // 128):
            pltpu.make_async_copy(
                ubuf, accum.at[pl.ds(s * ZROWS_PER_SUB + z * 128, 128)],
                gsem.at[0]).wait()

        plsc.subcore_barrier()

        def prefetch(p, ch):
            row0 = s * ROWS_PER_SUB + ch * CH
            pltpu.async_copy(src_hbm.at[pl.ds(row0, CH)], srcb.at[p], isem.at[p])
            pltpu.async_copy(dst_hbm.at[pl.ds(row0, CH)], sidxb.at[p], isem.at[p])
            pltpu.async_copy(typ_hbm.at[pl.ds(row0, CH)], typb.at[p], isem.at[p])

        def wait_idx(p):
            for _ in range(3):
                pltpu.make_async_copy(src_hbm.at[pl.ds(0, CH)], srcb.at[p],
                                      isem.at[p]).wait()

        def drain(slot, p, row):
            pltpu.make_async_copy(xa_hbm.at[srcb.at[p, row]], gbuf.at[slot],
                                  gsem.at[slot]).wait()

            @pl.loop(0, 128)
            def _(rr):
                for q in range(H // 32):
                    w = plsc.bitcast(gbuf[slot, rr, pl.ds(32 * q, 32)],
                                     jnp.int32)
                    lo = plsc.bitcast(lax.shift_left(w, 16), jnp.float32)
                    hi = plsc.bitcast(
                        lax.bitwise_and(w, jnp.int32(-65536)), jnp.float32)
                    ubuf[rr, pl.ds(32 * q, 16)] = lo
                    ubuf[rr, pl.ds(32 * q + 16, 16)] = hi

            pltpu.sync_copy(ubuf, accum.at[sidxb.at[p, row]], add=True)

        def chunk_body(p):
            wait_idx(p)
            for r in range(CH):
                @pl.loop(0, 128, step=16)
                def _(k):
                    srcb[p, r, pl.ds(k, 16)] = srcb[p, r, pl.ds(k, 16)] + coff
                    sidxb[p, r, pl.ds(k, 16)] = (
                        sidxb[p, r, pl.ds(k, 16)]
                        + typb[p, r, pl.ds(k, 16)] * N)
                if r >= RING:
                    drain(r % RING, p, r - RING)
                pltpu.async_copy(xa_hbm.at[srcb.at[p, r]],
                                 gbuf.at[r % RING], gsem.at[r % RING])
            for r in range(CH - RING, CH):
                drain(r % RING, p, r)

        prefetch(0, 0)

        @pl.loop(0, NCHUNK // 2)
        def _(ch2):
            prefetch(1, 2 * ch2 + 1)
            chunk_body(0)

            @pl.when(ch2 < NCHUNK // 2 - 1)
            def _():
                prefetch(0, 2 * ch2 + 2)

            chunk_body(1)

        plsc.subcore_barrier()
        pltpu.sync_copy(
            accum.at[pl.ds(s * ZROWS_PER_SUB, ZROWS_PER_SUB)],
            s_hbm.at[c, pl.ds(s * ZROWS_PER_SUB, ZROWS_PER_SUB)])

    return sc_kernel(xa, srcp, dstp, typp)


def _sc_counts(dstp, typp):
    mesh = plsc.VectorSubcoreMesh(core_axis_name="c", subcore_axis_name="s")

    @pl.kernel(
        out_type=jax.ShapeDtypeStruct((NCORE, AROWS, 16), jnp.float32),
        mesh=mesh,
        scratch_types=[
            pltpu.VMEM((CROWS_PER_SUB, 128), jnp.int32),
            pltpu.VMEM((CROWS_PER_SUB, 128), jnp.int32),
            pltpu.VMEM((128, 16), jnp.float32),
            pltpu.VMEM((128, 16), jnp.float32),
            pltpu.VMEM_SHARED((AROWS, 16), jnp.float32),
            pltpu.SemaphoreType.DMA,
            pltpu.SemaphoreType.DMA,
        ],
        compiler_params=_SC_PARAMS,
    )
    def sc_kernel(dst_hbm, typ_hbm, c_hbm, cidx_i, typ_i, ones_b, zero_b, accum,
                  isem, ssem):
        c = lax.axis_index("c")
        s = lax.axis_index("s")
        row0 = (c * NSUB + s) * CROWS_PER_SUB

        pltpu.async_copy(dst_hbm.at[pl.ds(row0, CROWS_PER_SUB)], cidx_i, isem)
        pltpu.async_copy(typ_hbm.at[pl.ds(row0, CROWS_PER_SUB)], typ_i, isem)

        @pl.loop(0, 128)
        def _(r):
            ones_b[r, pl.ds(0, 16)] = jnp.ones((16,), jnp.float32)
            zero_b[r, pl.ds(0, 16)] = jnp.zeros((16,), jnp.float32)

        for z in range(ZROWS_PER_SUB // 128):
            pltpu.async_copy(
                zero_b, accum.at[pl.ds(s * ZROWS_PER_SUB + z * 128, 128)], ssem)
        for z in range(ZROWS_PER_SUB // 128):
            pltpu.make_async_copy(
                zero_b, accum.at[pl.ds(s * ZROWS_PER_SUB + z * 128, 128)],
                ssem).wait()

        for _ in range(2):
            pltpu.make_async_copy(dst_hbm.at[pl.ds(0, CROWS_PER_SUB)], cidx_i,
                                  isem).wait()

        plsc.subcore_barrier()

        @pl.loop(0, CROWS_PER_SUB)
        def _(g):
            @pl.loop(0, 128, step=16)
            def _(k):
                cidx_i[g, pl.ds(k, 16)] = (
                    cidx_i[g, pl.ds(k, 16)] + typ_i[g, pl.ds(k, 16)] * N)
            pltpu.async_copy(ones_b, accum.at[cidx_i.at[g]], ssem, add=True)

        @pl.loop(0, CROWS_PER_SUB)
        def _(g):
            pltpu.make_async_copy(ones_b, accum.at[cidx_i.at[g]], ssem).wait()

        plsc.subcore_barrier()
        pltpu.sync_copy(
            accum.at[pl.ds(s * ZROWS_PER_SUB, ZROWS_PER_SUB)],
            c_hbm.at[c, pl.ds(s * ZROWS_PER_SUB, ZROWS_PER_SUB)])

    return sc_kernel(dstp, typp)


def _tc_combine(x, S, C, root, weight, bias2d):
    B = 1000
    grid = (N // B,)
    NB = N // B

    def tc_body(x_ref, s00, s10, s01, s11, c00, c10, c01, c11,
                root_ref, w_ref, b_ref, o_ref):
        cnt0 = c00[0, :, 0:1] + c10[0, :, 0:1]
        cnt1 = c01[0, :, 0:1] + c11[0, :, 0:1]
        inv0 = 1.0 / jnp.clip(cnt0, 1.0, None)
        inv1 = 1.0 / jnp.clip(cnt1, 1.0, None)
        m0 = jnp.concatenate([s00[0], s10[0]], axis=1) * inv0
        m1 = jnp.concatenate([s01[0], s11[0]], axis=1) * inv1
        out = jnp.dot(x_ref[...], root_ref[...],
                      preferred_element_type=jnp.float32)
        out = out + jnp.dot(m0, w_ref[0], preferred_element_type=jnp.float32)
        out = out + jnp.dot(m1, w_ref[1], preferred_element_type=jnp.float32)
        o_ref[...] = out + b_ref[...]

    sblk = (1, B, H)
    cblk = (1, B, 16)
    return pl.pallas_call(
        tc_body,
        grid=grid,
        in_specs=[
            pl.BlockSpec((B, D), lambda i: (i, 0)),
            pl.BlockSpec(sblk, lambda i: (0, i, 0)),
            pl.BlockSpec(sblk, lambda i: (1, i, 0)),
            pl.BlockSpec(sblk, lambda i: (0, i + NB, 0)),
            pl.BlockSpec(sblk, lambda i: (1, i + NB, 0)),
            pl.BlockSpec(cblk, lambda i: (0, i, 0)),
            pl.BlockSpec(cblk, lambda i: (1, i, 0)),
            pl.BlockSpec(cblk, lambda i: (0, i + NB, 0)),
            pl.BlockSpec(cblk, lambda i: (1, i + NB, 0)),
            pl.BlockSpec((D, D), lambda i: (0, 0)),
            pl.BlockSpec((2, D, D), lambda i: (0, 0, 0)),
            pl.BlockSpec((1, D), lambda i: (0, 0)),
        ],
        out_specs=pl.BlockSpec((B, D), lambda i: (i, 0)),
        out_shape=jax.ShapeDtypeStruct((N, D), jnp.float32),
    )(x, S, S, S, S, C, C, C, C, root, weight, bias2d)


def kernel(node_features, edge_index, edge_type, weight, root, bias):
    x = node_features
    src = edge_index[0]
    dst = edge_index[1]

    perm = []
    for q in range(H // 32):
        for k in range(16):
            perm.extend([32 * q + k, 32 * q + 16 + k])
    perm = jnp.array(perm, dtype=jnp.int32)
    xa = jnp.concatenate([x[:, :H], x[:, H:]], axis=0).astype(jnp.bfloat16)
    xa = xa[:, perm]

    pad = EP - E
    srcp = jnp.concatenate([src, jnp.zeros((pad,), jnp.int32)]).reshape(EROWS, 128)
    dstp = jnp.concatenate([dst, jnp.full((pad,), N, jnp.int32)]).reshape(EROWS, 128)
    typp = jnp.concatenate([edge_type, jnp.full((pad,), 1, jnp.int32)]).reshape(EROWS, 128)

    S = _sc_segment_sums(xa, srcp, dstp, typp)
    C = _sc_counts(dstp, typp)
    return _tc_combine(x, S, C, root, weight, bias.reshape(1, D))
